# Initial kernel scaffold; baseline (speedup 1.0000x reference)
import jax
import jax.numpy as jnp
from jax.experimental import pallas as pl


def kernel(feats, edge_index, etypes, W1, w_ih1, w_hh1, b_ih1, b_hh1, W2, w_ih2, w_hh2, b_ih2, b_hh2):
    raise NotImplementedError("write your pallas kernel here")


# speedup vs baseline: 8.0576x; 8.0576x over previous
import functools

import jax
import jax.numpy as jnp
from jax import lax
from jax.experimental import pallas as pl
from jax.experimental.pallas import tpu as pltpu
from jax.experimental.pallas import tpu_sc as plsc

N = 10000
E = 320000
D = 128
NT = 2
NC = 2
NS = 16
NW = NC * NS
CHUNK = 128
NCHUNK = E // CHUNK
CH_PER_W = -(-NCHUNK // NW)
N_PAD = 10240
ROWS_PER_TILE = N_PAD // NS

MM_BLK = 2000
GRU_BLK = 1000


def _hw_body(h_ref, wt_ref, out_ref):
    out_ref[...] = jnp.dot(h_ref[...], wt_ref[0],
                           preferred_element_type=jnp.float32)


def _hw_table(h, Wt):
    nb = N // MM_BLK
    return pl.pallas_call(
        _hw_body,
        grid=(NT, nb),
        in_specs=[
            pl.BlockSpec((MM_BLK, D), lambda e, i: (i, 0)),
            pl.BlockSpec((1, D, D), lambda e, i: (e, 0, 0)),
        ],
        out_specs=pl.BlockSpec((MM_BLK, D), lambda e, i: (e * nb + i, 0)),
        out_shape=jax.ShapeDtypeStruct((NT * N, D), jnp.float32),
    )(h, Wt)


def _sc_scatter_body(table_hbm, src_hbm, dst_hbm, et_hbm, zeros_hbm, out_hbm,
                     src_v, dst_v, et_v, idx_v, rows_v, acc_sh, sem):
    c = lax.axis_index("c")
    s = lax.axis_index("s")
    wid = c * NS + s

    pltpu.sync_copy(zeros_hbm, acc_sh.at[pl.ds(s * ROWS_PER_TILE, ROWS_PER_TILE)])
    plsc.subcore_barrier()

    def step(t, carry):
        ch = wid + t * NW

        @pl.when(ch < NCHUNK)
        def _():
            base = ch * CHUNK
            pltpu.sync_copy(src_hbm.at[pl.ds(base, CHUNK)], src_v)
            pltpu.sync_copy(dst_hbm.at[pl.ds(base, CHUNK)], dst_v)
            pltpu.sync_copy(et_hbm.at[pl.ds(base, CHUNK)], et_v)
            for j in range(CHUNK // 16):
                sl = pl.ds(j * 16, 16)
                idx_v[sl] = et_v[sl] * N + src_v[sl]
            pltpu.async_copy(table_hbm.at[idx_v], rows_v, sem).wait()
            pltpu.sync_copy(rows_v, acc_sh.at[dst_v], add=True)

        return carry

    lax.fori_loop(0, CH_PER_W, step, 0)
    plsc.subcore_barrier()

    sl = pl.ds(s * ROWS_PER_TILE, ROWS_PER_TILE)
    pltpu.sync_copy(acc_sh.at[sl], out_hbm.at[c].at[sl])


@functools.cache
def _get_sc_scatter():
    return pl.kernel(
        _sc_scatter_body,
        out_type=jax.ShapeDtypeStruct((NC, N_PAD, D), jnp.float32),
        mesh=plsc.VectorSubcoreMesh(core_axis_name="c", subcore_axis_name="s"),
        scratch_types=[
            pltpu.VMEM((CHUNK,), jnp.int32),
            pltpu.VMEM((CHUNK,), jnp.int32),
            pltpu.VMEM((CHUNK,), jnp.int32),
            pltpu.VMEM((CHUNK,), jnp.int32),
            pltpu.VMEM((CHUNK, D), jnp.float32),
            pltpu.VMEM_SHARED((N_PAD, D), jnp.float32),
            pltpu.SemaphoreType.DMA,
        ],
    )


def _sc_scatter(table, src, dst, etypes, zeros):
    return _get_sc_scatter()(table, src, dst, etypes, zeros)


def _sigmoid(x):
    return 1.0 / (1.0 + jnp.exp(-x))


def _gru_body(p_ref, h_ref, wih_ref, whh_ref, bih_ref, bhh_ref, out_ref):
    a = p_ref[0] + p_ref[1]
    h = h_ref[...]
    gi = jnp.dot(a, wih_ref[...], preferred_element_type=jnp.float32) + bih_ref[...]
    gh = jnp.dot(h, whh_ref[...], preferred_element_type=jnp.float32) + bhh_ref[...]
    i_r, i_z, i_n = gi[:, :D], gi[:, D:2 * D], gi[:, 2 * D:]
    h_r, h_z, h_n = gh[:, :D], gh[:, D:2 * D], gh[:, 2 * D:]
    r = _sigmoid(i_r + h_r)
    z = _sigmoid(i_z + h_z)
    n = jnp.tanh(i_n + r * h_n)
    out_ref[...] = (1.0 - z) * n + z * h


def _gru(partials, h, wih_t, whh_t, bih, bhh):
    nb = N // GRU_BLK
    return pl.pallas_call(
        _gru_body,
        grid=(nb,),
        in_specs=[
            pl.BlockSpec((NC, GRU_BLK, D), lambda i: (0, i, 0)),
            pl.BlockSpec((GRU_BLK, D), lambda i: (i, 0)),
            pl.BlockSpec((D, 3 * D), lambda i: (0, 0)),
            pl.BlockSpec((D, 3 * D), lambda i: (0, 0)),
            pl.BlockSpec((1, 3 * D), lambda i: (0, 0)),
            pl.BlockSpec((1, 3 * D), lambda i: (0, 0)),
        ],
        out_specs=pl.BlockSpec((GRU_BLK, D), lambda i: (i, 0)),
        out_shape=jax.ShapeDtypeStruct((N, D), jnp.float32),
    )(partials, h, wih_t, whh_t, bih, bhh)


def kernel(feats, edge_index, etypes,
           W1, w_ih1, w_hh1, b_ih1, b_hh1,
           W2, w_ih2, w_hh2, b_ih2, b_hh2):
    src = edge_index[0]
    dst = edge_index[1]
    zeros = jnp.zeros((ROWS_PER_TILE, D), jnp.float32)

    h = feats
    outs = []
    for (W, wih, whh, bih, bhh) in (
            (W1, w_ih1, w_hh1, b_ih1, b_hh1),
            (W2, w_ih2, w_hh2, b_ih2, b_hh2)):
        Wt = jnp.swapaxes(W, 1, 2)
        wih_t = wih.T
        whh_t = whh.T
        bih2 = bih[None, :]
        bhh2 = bhh[None, :]
        for _ in range(2):
            table = _hw_table(h, Wt)
            partials = _sc_scatter(table, src, dst, etypes, zeros)
            h = _gru(partials, h, wih_t, whh_t, bih2, bhh2)
        outs.append(h)
    return jnp.stack(outs, axis=0)
